# Initial kernel scaffold; baseline (speedup 1.0000x reference)
import jax
import jax.numpy as jnp
from jax.experimental import pallas as pl


def kernel(x1, x2):
    raise NotImplementedError("write your pallas kernel here")


# speedup vs baseline: 1.0228x; 1.0228x over previous
import jax
import jax.numpy as jnp
from jax.experimental import pallas as pl
from jax.experimental.pallas import tpu as pltpu

MARGIN = 0.3
PD_EPS = 1e-6
B = 1024
D = 2048
BM = 256
NI = B // BM


def _triplet_kernel(a_ref, p_ref, out_ref, pn_ref, colp_ref):
    i = pl.program_id(0)

    @pl.when(i == 0)
    def _init():
        p = p_ref[...]
        nrm = jnp.sqrt(jnp.sum(p * p, axis=1, keepdims=True))
        pn = p / jnp.maximum(nrm, 1e-12)
        pn_ref[...] = pn
        colp_ref[...] = (jnp.sum(pn * pn, axis=1, keepdims=True)
                         - (2.0 * PD_EPS) * jnp.sum(pn, axis=1, keepdims=True))
        out_ref[...] = jnp.zeros_like(out_ref)

    a = a_ref[...]
    nrma = jnp.sqrt(jnp.sum(a * a, axis=1, keepdims=True))
    an = a / jnp.maximum(nrma, 1e-12)

    cross = jax.lax.dot_general(
        pn_ref[...], an, (((1,), (1,)), ((), ())),
        preferred_element_type=jnp.float32)
    h = colp_ref[...] - 2.0 * cross

    rowj = jax.lax.broadcasted_iota(jnp.int32, (B, BM), 0)
    coli = jax.lax.broadcasted_iota(jnp.int32, (B, BM), 1) + i * BM
    diag = rowj == coli

    hneg = jnp.min(jnp.where(diag, jnp.float32(3.0e38), h), axis=0,
                   keepdims=True)
    hpos = jnp.sum(jnp.where(diag, h, 0.0), axis=0, keepdims=True)
    lv = jnp.maximum(hpos - hneg + MARGIN, 0.0) * (1.0 / B)
    out_ref[...] += jnp.sum(lv, axis=1, keepdims=True)


def kernel(x1, x2):
    a2 = x1.reshape(B, 2 * D)
    p2 = x2.reshape(B, 2 * D)
    out = pl.pallas_call(
        _triplet_kernel,
        grid=(NI,),
        in_specs=[
            pl.BlockSpec((BM, D), lambda i: (i, 0)),
            pl.BlockSpec((B, D), lambda i: (0, 1)),
        ],
        out_specs=pl.BlockSpec((1, 1), lambda i: (0, 0)),
        out_shape=jax.ShapeDtypeStruct((1, 1), jnp.float32),
        scratch_shapes=[
            pltpu.VMEM((B, D), jnp.float32),
            pltpu.VMEM((B, 1), jnp.float32),
        ],
        compiler_params=pltpu.CompilerParams(
            dimension_semantics=("arbitrary",),
        ),
    )(a2, p2)
    return out[0, 0]
